# baseline (speedup 1.0000x reference)
import jax
import jax.numpy as jnp
from jax.experimental import pallas as pl
from jax.experimental.pallas import tpu as pltpu

_BLOCK = 512


def _router_kernel(x_ref, w_ref, o_ref):
    logits = jax.lax.dot_general(
        x_ref[...],
        w_ref[...],
        dimension_numbers=(((1,), (1,)), ((), ())),
        preferred_element_type=jnp.float32,
    )
    m = jnp.max(logits, axis=-1, keepdims=True)
    e = jnp.exp(logits - m)
    o_ref[...] = e / jnp.sum(e, axis=-1, keepdims=True)


def kernel(x, W):
    n_tokens, in_dim = x.shape
    n_experts = W.shape[0]
    return pl.pallas_call(
        _router_kernel,
        grid=(n_tokens // _BLOCK,),
        in_specs=[
            pl.BlockSpec((_BLOCK, in_dim), lambda i: (i, 0)),
            pl.BlockSpec((n_experts, in_dim), lambda i: (0, 0)),
        ],
        out_specs=pl.BlockSpec((_BLOCK, n_experts), lambda i: (i, 0)),
        out_shape=jax.ShapeDtypeStruct((n_tokens, n_experts), jnp.float32),
        compiler_params=pltpu.CompilerParams(
            dimension_semantics=("parallel",)
        ),
    )(x, W)


# speedup vs baseline: 1.1681x; 1.1681x over previous
import jax
import jax.numpy as jnp
from jax.experimental import pallas as pl
from jax.experimental.pallas import tpu as pltpu

_BLOCK = 2048


def _router_kernel(x_ref, w_ref, o_ref):
    logits = jax.lax.dot_general(
        x_ref[...],
        w_ref[...],
        dimension_numbers=(((1,), (1,)), ((), ())),
        preferred_element_type=jnp.float32,
    )
    m = jnp.max(logits, axis=-1, keepdims=True)
    e = jnp.exp(logits - m)
    o_ref[...] = e / jnp.sum(e, axis=-1, keepdims=True)


def kernel(x, W):
    n_tokens, in_dim = x.shape
    n_experts = W.shape[0]
    return pl.pallas_call(
        _router_kernel,
        grid=(n_tokens // _BLOCK,),
        in_specs=[
            pl.BlockSpec((_BLOCK, in_dim), lambda i: (i, 0)),
            pl.BlockSpec((n_experts, in_dim), lambda i: (0, 0)),
        ],
        out_specs=pl.BlockSpec((_BLOCK, n_experts), lambda i: (i, 0)),
        out_shape=jax.ShapeDtypeStruct((n_tokens, n_experts), jnp.float32),
        compiler_params=pltpu.CompilerParams(
            dimension_semantics=("parallel",)
        ),
    )(x, W)


# speedup vs baseline: 1.1814x; 1.0113x over previous
import jax
import jax.numpy as jnp
from jax.experimental import pallas as pl
from jax.experimental.pallas import tpu as pltpu

_BLOCK = 1024


def _router_kernel(x_ref, w_ref, o_ref):
    logits = jax.lax.dot_general(
        x_ref[...],
        w_ref[...],
        dimension_numbers=(((1,), (1,)), ((), ())),
        preferred_element_type=jnp.float32,
    )
    m = jnp.max(logits, axis=-1, keepdims=True)
    e = jnp.exp(logits - m)
    o_ref[...] = e / jnp.sum(e, axis=-1, keepdims=True)


def kernel(x, W):
    n_tokens, in_dim = x.shape
    n_experts = W.shape[0]
    return pl.pallas_call(
        _router_kernel,
        grid=(n_tokens // _BLOCK,),
        in_specs=[
            pl.BlockSpec((_BLOCK, in_dim), lambda i: (i, 0)),
            pl.BlockSpec((n_experts, in_dim), lambda i: (0, 0)),
        ],
        out_specs=pl.BlockSpec((_BLOCK, n_experts), lambda i: (i, 0)),
        out_shape=jax.ShapeDtypeStruct((n_tokens, n_experts), jnp.float32),
        compiler_params=pltpu.CompilerParams(
            dimension_semantics=("parallel",)
        ),
    )(x, W)
